# Initial kernel scaffold; baseline (speedup 1.0000x reference)
import jax
import jax.numpy as jnp
from jax.experimental import pallas as pl


def kernel(z, embedding_weight):
    raise NotImplementedError("write your pallas kernel here")


# speedup vs baseline: 10.3141x; 10.3141x over previous
import functools

import jax
import jax.numpy as jnp
from jax import lax
from jax.experimental import pallas as pl
from jax.experimental.pallas import tpu as pltpu
from jax.experimental.pallas import tpu_sc as plsc

_Z_DIM = 32
_K = 8192
_N = 8192
_BETA = 0.25

_TN = 512
_GRID = _N // _TN

_NC = 2
_NS = 16
_NW = _NC * _NS
_BPW = _N // _NW
_IDX_CH = 128


_HK = _K // 2


def _dist_argmin_body(z_ref, ebf_ref, csz_ref, cse_ref, idx_ref, loss_ref):
    i = pl.program_id(0)
    z = z_ref[...]
    csz = csz_ref[0][...].reshape(_TN, 1)

    def chunk(c0):
        eb = ebf_ref[c0:c0 + _HK, :]
        mm = lax.dot_general(z, eb, (((1,), (1,)), ((), ())),
                             preferred_element_type=jnp.float32)
        d = (csz + cse_ref[:, c0:c0 + _HK]) - 2.0 * mm
        minv = jnp.min(d, axis=1)
        iota = lax.broadcasted_iota(jnp.int32, (_TN, _HK), 1)
        idx = jnp.min(jnp.where(d == minv[:, None], iota, _HK), axis=1)
        return minv, idx + c0

    minv_a, idx_a = chunk(0)
    minv_b, idx_b = chunk(_HK)
    min_a_bf = minv_a.astype(jnp.bfloat16).astype(jnp.float32)
    take_b = minv_b < min_a_bf
    idx = jnp.where(take_b, idx_b, idx_a)
    minv = jnp.where(take_b, minv_b, minv_a)
    idx_ref[0, 0, :] = idx
    part = jnp.sum(minv)
    prev = jnp.where(i == 0, jnp.float32(0.0), loss_ref[0, 0])
    acc = prev + part
    scale = jnp.float32((1.0 + _BETA) / (_N * _Z_DIM))
    loss_ref[0, 0] = jnp.where(i == _GRID - 1, acc * scale, acc)


def _dist_argmin(z, ebf, csz, cse):
    return pl.pallas_call(
        _dist_argmin_body,
        grid=(_GRID,),
        in_specs=[
            pl.BlockSpec((_TN, _Z_DIM), lambda i: (i, 0)),
            pl.BlockSpec((_K, _Z_DIM), lambda i: (0, 0)),
            pl.BlockSpec((1, 1, _TN), lambda i: (i, 0, 0)),
            pl.BlockSpec((1, _K), lambda i: (0, 0)),
        ],
        out_specs=[
            pl.BlockSpec((1, 1, _TN), lambda i: (i, 0, 0)),
            pl.BlockSpec((1, 1), lambda i: (0, 0),
                         memory_space=pltpu.SMEM),
        ],
        out_shape=[
            jax.ShapeDtypeStruct((_GRID, 1, _TN), jnp.int32),
            jax.ShapeDtypeStruct((1, 1), jnp.float32),
        ],
    )(z, ebf, csz, cse)


@functools.lru_cache(maxsize=1)
def _make_sc_gather():
    mesh = plsc.VectorSubcoreMesh(core_axis_name="c", subcore_axis_name="s")

    @functools.partial(
        pl.kernel,
        mesh=mesh,
        out_type=jax.ShapeDtypeStruct((_N, _Z_DIM), jnp.float32),
        scratch_types=[
            pltpu.VMEM((_BPW // _IDX_CH, _IDX_CH), jnp.int32),
            pltpu.VMEM((_BPW, _Z_DIM), jnp.float32),
            pltpu.SemaphoreType.DMA,
        ],
        compiler_params=pltpu.CompilerParams(use_tc_tiling_on_sc=False),
    )
    def _sc_gather(table_hbm, idx_hbm, out_hbm, idx_v, rows_v, sem):
        wid = lax.axis_index("s") * _NC + lax.axis_index("c")
        nrow = _BPW // _IDX_CH
        pltpu.sync_copy(idx_hbm.at[pl.ds(wid * nrow, nrow)], idx_v)
        for j in range(nrow):
            pltpu.async_copy(
                table_hbm.at[idx_v.at[j]],
                rows_v.at[pl.ds(j * _IDX_CH, _IDX_CH)],
                sem,
            ).wait()
        pltpu.sync_copy(rows_v, out_hbm.at[pl.ds(wid * _BPW, _BPW)])

    return _sc_gather


def kernel(z, embedding_weight):
    z_flat = z
    csz = jnp.sum(z_flat ** 2, axis=1, keepdims=True)
    cse = jnp.sum(embedding_weight ** 2, axis=1)
    ebf = embedding_weight.astype(jnp.bfloat16)
    idx_blk, loss_blk = _dist_argmin(
        z_flat, ebf,
        csz.reshape(_GRID, 1, _TN), cse.reshape(1, _K))
    idx = idx_blk.reshape(_N)
    z_q = _make_sc_gather()(embedding_weight,
                            idx.reshape(_N // _IDX_CH, _IDX_CH))
    embedding_loss = loss_blk.reshape(())
    z_q_st = z_flat + lax.stop_gradient(z_q - z_flat)
    return z_q_st, idx.reshape(_N, 1), embedding_loss
